# baseline (speedup 1.0000x reference)
import functools

import jax
import jax.numpy as jnp
from jax import lax
from jax.experimental import pallas as pl
from jax.experimental.pallas import tpu as pltpu
from jax.experimental.pallas import tpu_sc as plsc

BATCH = 4096
INPUT_DIM = 100000
OUTPUT_DIM = 64
NNZ = 204800

NC, NS = 2, 16
NW = NC * NS
RPW = BATCH // NW
GK = 128
NG = 8
CK = GK * NG

_mesh = plsc.VectorSubcoreMesh(
    core_axis_name="c", subcore_axis_name="s", num_cores=NC, num_subcores=NS
)


@functools.partial(
    pl.kernel,
    out_type=jax.ShapeDtypeStruct((BATCH, OUTPUT_DIM), jnp.float32),
    mesh=_mesh,
    scratch_types=[
        pltpu.VMEM((NW + 16,), jnp.int32),
        pltpu.VMEM((NW + 16,), jnp.int32),
        pltpu.VMEM((OUTPUT_DIM,), jnp.float32),
        pltpu.VMEM((2 * CK,), jnp.int32),
        pltpu.VMEM((2 * CK,), jnp.int32),
        pltpu.VMEM((2 * CK,), jnp.float32),
        pltpu.VMEM((CK, OUTPUT_DIM), jnp.float32),
        pltpu.VMEM((RPW + 1, OUTPUT_DIM), jnp.float32),
        pltpu.VMEM((OUTPUT_DIM,), jnp.float32),
        pltpu.SemaphoreType.DMA,
        pltpu.SemaphoreType.DMA,
        pltpu.SemaphoreType.DMA,
    ],
    compiler_params=pltpu.CompilerParams(use_tc_tiling_on_sc=False),
)
def _embedding_bag(val_hbm, row_hbm, col_hbm, w_hbm, bias_hbm, cs_hbm, nch_hbm,
                   out_hbm, cs_v, nch_v, bias_v, idx_v, row_v, valv, rows_v,
                   acc_v, snap_v, gsem, gsem2, msem):
    wid = lax.axis_index("s") * NC + lax.axis_index("c")
    base_row = wid * RPW

    pltpu.sync_copy(cs_hbm, cs_v)
    pltpu.sync_copy(nch_hbm, nch_v)
    pltpu.sync_copy(bias_hbm, bias_v)
    start = cs_v[pl.ds(wid, 16)][0]
    nch = nch_v[pl.ds(wid, 16)][0]

    zeros = jnp.zeros((16,), jnp.float32)

    def zero_row(r, carry):
        for j in range(OUTPUT_DIM // 16):
            acc_v[r, pl.ds(16 * j, 16)] = zeros
        return carry

    lax.fori_loop(0, RPW + 1, zero_row, 0)

    H = CK // 2
    NGH = NG // 2

    def issue_meta(ci, slot_off):
        g = pl.multiple_of(start + ci * CK, CK)
        so = pl.multiple_of(slot_off, CK)
        pltpu.async_copy(col_hbm.at[pl.ds(g, CK)], idx_v.at[pl.ds(so, CK)], msem)
        pltpu.async_copy(row_hbm.at[pl.ds(g, CK)], row_v.at[pl.ds(so, CK)], msem)
        pltpu.async_copy(val_hbm.at[pl.ds(g, CK)], valv.at[pl.ds(so, CK)], msem)

    def wait_meta():
        pltpu.make_async_copy(col_hbm.at[pl.ds(0, CK)], idx_v.at[pl.ds(0, CK)], msem).wait()
        pltpu.make_async_copy(row_hbm.at[pl.ds(0, CK)], row_v.at[pl.ds(0, CK)], msem).wait()
        pltpu.make_async_copy(val_hbm.at[pl.ds(0, CK)], valv.at[pl.ds(0, CK)], msem).wait()

    def issue_wave(slot_off, wave, sem):
        for j in range(NGH):
            off = wave * H + j * GK
            pltpu.async_copy(
                w_hbm.at[idx_v.at[pl.ds(pl.multiple_of(slot_off + off, GK), GK)]],
                rows_v.at[pl.ds(off, GK)],
                sem,
            )

    def wait_wave(sem):
        pltpu.make_async_copy(
            w_hbm.at[pl.ds(0, H)], rows_v.at[pl.ds(0, H)], sem
        ).wait()

    def compute_half(slot_off, rows_off, st0):
        def per_group(i16, st):
            pr, a0, a1, a2, a3 = st
            rbase = rows_off + i16 * 16
            mbase = slot_off + rbase
            rv = row_v[pl.ds(mbase, 16)] - base_row
            rv = jnp.where((rv >= 0) & (rv < RPW), rv, RPW)
            vv = valv[pl.ds(mbase, 16)]
            for k in range(16):
                r = rv[k]
                v = vv[k]
                new = r != pr

                @pl.when(new)
                def _(pr=pr, regs=(a0, a1, a2, a3)):
                    for j in range(OUTPUT_DIM // 16):
                        sj = snap_v[pl.ds(16 * j, 16)]
                        acc_v[pr, pl.ds(16 * j, 16)] += regs[j] - sj
                        snap_v[pl.ds(16 * j, 16)] = regs[j]

                a0 = a0 + v * rows_v[rbase + k, pl.ds(0, 16)]
                a1 = a1 + v * rows_v[rbase + k, pl.ds(16, 16)]
                a2 = a2 + v * rows_v[rbase + k, pl.ds(32, 16)]
                a3 = a3 + v * rows_v[rbase + k, pl.ds(48, 16)]
                pr = r
            return pr, a0, a1, a2, a3

        return lax.fori_loop(0, H // 16, per_group, st0)

    @pl.when(nch > 0)
    def _():
        g0 = pl.multiple_of(start, CK)
        pltpu.sync_copy(col_hbm.at[pl.ds(g0, CK)], idx_v.at[pl.ds(0, CK)])
        pltpu.sync_copy(row_hbm.at[pl.ds(g0, CK)], row_v.at[pl.ds(0, CK)])
        pltpu.sync_copy(val_hbm.at[pl.ds(g0, CK)], valv.at[pl.ds(0, CK)])
        issue_wave(0, 0, gsem)

        @pl.when(nch > 1)
        def _():
            issue_meta(1, CK)

        def do_chunk(ci, carry):
            slot = lax.rem(ci, 2)
            so = pl.multiple_of(slot * CK, CK)
            nso = pl.multiple_of(CK - slot * CK, CK)
            issue_wave(so, 1, gsem2)
            for j in range(OUTPUT_DIM // 16):
                snap_v[pl.ds(16 * j, 16)] = zeros
            wait_wave(gsem)
            st = compute_half(so, 0, (jnp.int32(RPW), zeros, zeros, zeros, zeros))

            @pl.when(ci + 1 < nch)
            def _():
                wait_meta()
                issue_wave(nso, 0, gsem)

            wait_wave(gsem2)
            st = compute_half(so, H, st)

            @pl.when(ci + 2 < nch)
            def _():
                issue_meta(ci + 2, so)

            pr, *regs = st
            for j in range(OUTPUT_DIM // 16):
                plsc.addupdate(
                    acc_v.at[pr, pl.ds(16 * j, 16)],
                    regs[j] - snap_v[pl.ds(16 * j, 16)],
                )
            return carry

        lax.fori_loop(0, nch, do_chunk, 0)

    def finish_row(r, carry):
        for j in range(OUTPUT_DIM // 16):
            x = acc_v[r, pl.ds(16 * j, 16)] + bias_v[pl.ds(16 * j, 16)]
            acc_v[r, pl.ds(16 * j, 16)] = jnp.maximum(x, 0.0)
        return carry

    lax.fori_loop(0, RPW, finish_row, 0)
    pltpu.sync_copy(acc_v.at[pl.ds(0, RPW)], out_hbm.at[pl.ds(base_row, RPW)])


def kernel(values, row_idx, col_idx, weights, bias):
    edges = jnp.arange(0, BATCH + 1, RPW, dtype=jnp.int32)
    bounds = jnp.searchsorted(row_idx, edges).astype(jnp.int32)
    cs = (bounds[:-1] // CK) * CK
    ce = ((bounds[1:] + (CK - 1)) // CK) * CK
    nch = (ce - cs) // CK
    cs = jnp.pad(cs, (0, 16))
    nch = jnp.pad(nch, (0, 16))
    return _embedding_bag(values, row_idx, col_idx, weights, bias, cs, nch)


# speedup vs baseline: 1.4058x; 1.4058x over previous
import functools

import numpy as np

import jax
import jax.numpy as jnp
from jax import lax
from jax.experimental import pallas as pl
from jax.experimental.pallas import tpu as pltpu
from jax.experimental.pallas import tpu_sc as plsc

BATCH = 4096
INPUT_DIM = 100000
OUTPUT_DIM = 64
NNZ = 204800

NC, NS = 2, 16
NW = NC * NS
RPW = BATCH // NW
GK = 128
NG = 8
CK = GK * NG

_mesh = plsc.VectorSubcoreMesh(
    core_axis_name="c", subcore_axis_name="s", num_cores=NC, num_subcores=NS
)


@functools.partial(
    pl.kernel,
    out_type=jax.ShapeDtypeStruct((BATCH, OUTPUT_DIM), jnp.float32),
    mesh=_mesh,
    scratch_types=[
        pltpu.VMEM((NW + 16,), jnp.int32),
        pltpu.VMEM((NW + 16,), jnp.int32),
        pltpu.VMEM((OUTPUT_DIM,), jnp.float32),
        pltpu.VMEM((2 * CK,), jnp.int32),
        pltpu.VMEM((2 * CK,), jnp.int32),
        pltpu.VMEM((2 * CK,), jnp.float32),
        pltpu.VMEM((CK, OUTPUT_DIM), jnp.bfloat16),
        pltpu.VMEM((RPW + 1, OUTPUT_DIM), jnp.float32),
        pltpu.SemaphoreType.DMA,
        pltpu.SemaphoreType.DMA,
        pltpu.SemaphoreType.DMA,
    ],
    compiler_params=pltpu.CompilerParams(
        use_tc_tiling_on_sc=False, needs_layout_passes=False
    ),
)
def _embedding_bag(val_hbm, row_hbm, col_hbm, w_hbm, bias_hbm, cs_hbm, nch_hbm,
                   out_hbm, cs_v, nch_v, bias_v, idx_v, row_v, valv, rows_v,
                   acc_v, gsem, gsem2, msem):
    wid = lax.axis_index("s") * NC + lax.axis_index("c")
    base_row = wid * RPW

    pltpu.sync_copy(cs_hbm, cs_v)
    pltpu.sync_copy(nch_hbm, nch_v)
    pltpu.sync_copy(bias_hbm, bias_v)
    start = cs_v[pl.ds(wid, 16)][0]
    nch = nch_v[pl.ds(wid, 16)][0]

    zeros = jnp.zeros((16,), jnp.float32)

    def zero_row(r, carry):
        for j in range(OUTPUT_DIM // 16):
            acc_v[r, pl.ds(16 * j, 16)] = zeros
        return carry

    lax.fori_loop(0, RPW + 1, zero_row, 0)

    H = CK // 2
    NGH = NG // 2

    def issue_meta(ci, slot_off):
        g = pl.multiple_of(start + ci * CK, CK)
        so = pl.multiple_of(slot_off, CK)
        pltpu.async_copy(col_hbm.at[pl.ds(g, CK)], idx_v.at[pl.ds(so, CK)], msem)
        pltpu.async_copy(row_hbm.at[pl.ds(g, CK)], row_v.at[pl.ds(so, CK)], msem)
        pltpu.async_copy(val_hbm.at[pl.ds(g, CK)], valv.at[pl.ds(so, CK)], msem)

    def wait_meta():
        pltpu.make_async_copy(col_hbm.at[pl.ds(0, CK)], idx_v.at[pl.ds(0, CK)], msem).wait()
        pltpu.make_async_copy(row_hbm.at[pl.ds(0, CK)], row_v.at[pl.ds(0, CK)], msem).wait()
        pltpu.make_async_copy(val_hbm.at[pl.ds(0, CK)], valv.at[pl.ds(0, CK)], msem).wait()

    def issue_wave(slot_off, wave, sem):
        for j in range(NGH):
            off = wave * H + j * GK
            pltpu.async_copy(
                w_hbm.at[idx_v.at[pl.ds(pl.multiple_of(slot_off + off, GK), GK)]],
                rows_v.at[pl.ds(off, GK)],
                sem,
            )

    def wait_wave(sem):
        pltpu.make_async_copy(
            w_hbm.at[pl.ds(0, H)], rows_v.at[pl.ds(0, H)], sem
        ).wait()

    def compute_half(slot_off, rows_off, st0):
        def per_group(i16, st):
            pr, a0, a1, a2, a3 = st
            rbase = rows_off + i16 * 16
            mbase = slot_off + rbase
            rv = row_v[pl.ds(mbase, 16)] - base_row
            vv = valv[pl.ds(mbase, 16)]
            for k in range(16):
                r = rv[k]
                v = vv[k]
                r = jnp.where((r >= 0) & (r < RPW), r, RPW)
                new = r != pr

                @pl.when(new)
                def _(pr=pr, regs=(a0, a1, a2, a3)):
                    for j in range(OUTPUT_DIM // 16):
                        acc_v[pr, pl.ds(16 * j, 16)] += regs[j]

                keep = jnp.where(new, 0.0, 1.0)
                w01 = plsc.unpack(
                    rows_v[rbase + k, pl.ds(0, 32)],
                    format=plsc.PackFormat.INTERLEAVED,
                )
                w23 = plsc.unpack(
                    rows_v[rbase + k, pl.ds(32, 32)],
                    format=plsc.PackFormat.INTERLEAVED,
                )
                a0 = a0 * keep + v * w01[0]
                a1 = a1 * keep + v * w01[1]
                a2 = a2 * keep + v * w23[0]
                a3 = a3 * keep + v * w23[1]
                pr = r
            return pr, a0, a1, a2, a3

        return lax.fori_loop(0, H // 16, per_group, st0)

    @pl.when(nch > 0)
    def _():
        g0 = pl.multiple_of(start, CK)
        pltpu.sync_copy(col_hbm.at[pl.ds(g0, CK)], idx_v.at[pl.ds(0, CK)])
        pltpu.sync_copy(row_hbm.at[pl.ds(g0, CK)], row_v.at[pl.ds(0, CK)])
        pltpu.sync_copy(val_hbm.at[pl.ds(g0, CK)], valv.at[pl.ds(0, CK)])
        issue_wave(0, 0, gsem)

        @pl.when(nch > 1)
        def _():
            issue_meta(1, CK)

        def do_chunk(ci, carry):
            slot = lax.rem(ci, 2)
            so = pl.multiple_of(slot * CK, CK)
            nso = pl.multiple_of(CK - slot * CK, CK)
            issue_wave(so, 1, gsem2)
            wait_wave(gsem)
            st = compute_half(so, 0, (jnp.int32(RPW), zeros, zeros, zeros, zeros))

            @pl.when(ci + 1 < nch)
            def _():
                wait_meta()
                issue_wave(nso, 0, gsem)

            wait_wave(gsem2)
            st = compute_half(so, H, st)

            @pl.when(ci + 2 < nch)
            def _():
                issue_meta(ci + 2, so)

            pr, *regs = st
            for j in range(OUTPUT_DIM // 16):
                plsc.addupdate(acc_v.at[pr, pl.ds(16 * j, 16)], regs[j])
            return carry

        lax.fori_loop(0, nch, do_chunk, 0)

    def finish_row(r, carry):
        for j in range(OUTPUT_DIM // 16):
            x = acc_v[r, pl.ds(16 * j, 16)] + bias_v[pl.ds(16 * j, 16)]
            acc_v[r, pl.ds(16 * j, 16)] = jnp.maximum(x, 0.0)
        return carry

    lax.fori_loop(0, RPW, finish_row, 0)
    pltpu.sync_copy(acc_v.at[pl.ds(0, RPW)], out_hbm.at[pl.ds(base_row, RPW)])


_PERM = np.asarray(
    [p for base in (0, 32) for pair in zip(range(base, base + 16),
                                           range(base + 16, base + 32))
     for p in pair],
    dtype=np.int32,
)


def kernel(values, row_idx, col_idx, weights, bias):
    weights = weights[:, _PERM].astype(jnp.bfloat16)
    edges = jnp.arange(0, BATCH + 1, RPW, dtype=jnp.int32)
    bounds = jnp.searchsorted(row_idx, edges).astype(jnp.int32)
    cs = (bounds[:-1] // CK) * CK
    ce = ((bounds[1:] + (CK - 1)) // CK) * CK
    nch = (ce - cs) // CK
    cs = jnp.pad(cs, (0, 16))
    nch = jnp.pad(nch, (0, 16))
    return _embedding_bag(values, row_idx, col_idx, weights, bias, cs, nch)


# speedup vs baseline: 1.6898x; 1.2020x over previous
import functools

import numpy as np

import jax
import jax.numpy as jnp
from jax import lax
from jax.experimental import pallas as pl
from jax.experimental.pallas import tpu as pltpu
from jax.experimental.pallas import tpu_sc as plsc

BATCH = 4096
INPUT_DIM = 100000
OUTPUT_DIM = 64
NNZ = 204800

NC, NS = 2, 16
NW = NC * NS
RPW = BATCH // NW
GK = 128
NG = 8
CK = GK * NG

_mesh = plsc.VectorSubcoreMesh(
    core_axis_name="c", subcore_axis_name="s", num_cores=NC, num_subcores=NS
)


@functools.partial(
    pl.kernel,
    out_type=jax.ShapeDtypeStruct((BATCH, OUTPUT_DIM), jnp.float32),
    mesh=_mesh,
    scratch_types=[
        pltpu.VMEM((NW + 16,), jnp.int32),
        pltpu.VMEM((NW + 16,), jnp.int32),
        pltpu.VMEM((OUTPUT_DIM,), jnp.float32),
        pltpu.VMEM((2 * CK,), jnp.int32),
        pltpu.VMEM((2 * CK,), jnp.int32),
        pltpu.VMEM((2 * CK,), jnp.float32),
        pltpu.VMEM((CK, OUTPUT_DIM), jnp.float32),
        pltpu.VMEM((RPW + 1, OUTPUT_DIM), jnp.float32),
        pltpu.SemaphoreType.DMA,
        pltpu.SemaphoreType.DMA,
        pltpu.SemaphoreType.DMA,
    ],
    compiler_params=pltpu.CompilerParams(
        use_tc_tiling_on_sc=False, needs_layout_passes=False
    ),
)
def _embedding_bag(val_hbm, row_hbm, col_hbm, w_hbm, bias_hbm, cs_hbm, nch_hbm,
                   out_hbm, cs_v, nch_v, bias_v, idx_v, row_v, valv, rows_v,
                   acc_v, gsem, gsem2, msem):
    wid = lax.axis_index("s") * NC + lax.axis_index("c")
    base_row = wid * RPW

    pltpu.sync_copy(cs_hbm, cs_v)
    pltpu.sync_copy(nch_hbm, nch_v)
    pltpu.sync_copy(bias_hbm, bias_v)
    start = cs_v[pl.ds(wid, 16)][0]
    nch = nch_v[pl.ds(wid, 16)][0]

    zeros = jnp.zeros((16,), jnp.float32)

    def zero_row(r, carry):
        for j in range(OUTPUT_DIM // 16):
            acc_v[r, pl.ds(16 * j, 16)] = zeros
        return carry

    lax.fori_loop(0, RPW + 1, zero_row, 0)

    H = CK // 2
    NGH = NG // 2

    def issue_meta(ci, slot_off):
        g = pl.multiple_of(start + ci * CK, GK)
        so = pl.multiple_of(slot_off, CK)
        pltpu.async_copy(col_hbm.at[pl.ds(g, CK)], idx_v.at[pl.ds(so, CK)], msem)
        pltpu.async_copy(row_hbm.at[pl.ds(g, CK)], row_v.at[pl.ds(so, CK)], msem)
        pltpu.async_copy(val_hbm.at[pl.ds(g, CK)], valv.at[pl.ds(so, CK)], msem)

    def wait_meta():
        pltpu.make_async_copy(col_hbm.at[pl.ds(0, CK)], idx_v.at[pl.ds(0, CK)], msem).wait()
        pltpu.make_async_copy(row_hbm.at[pl.ds(0, CK)], row_v.at[pl.ds(0, CK)], msem).wait()
        pltpu.make_async_copy(val_hbm.at[pl.ds(0, CK)], valv.at[pl.ds(0, CK)], msem).wait()

    def issue_wave(slot_off, wave, sem):
        for j in range(NGH):
            off = wave * H + j * GK
            pltpu.async_copy(
                w_hbm.at[idx_v.at[pl.ds(pl.multiple_of(slot_off + off, GK), GK)]],
                rows_v.at[pl.ds(off, GK)],
                sem,
            )

    def wait_wave(sem):
        pltpu.make_async_copy(
            w_hbm.at[pl.ds(0, H)], rows_v.at[pl.ds(0, H)], sem
        ).wait()

    def compute_half(slot_off, rows_off, pr0):
        def per_group(i16, pr):
            rbase = rows_off + i16 * 16
            mbase = slot_off + rbase
            rv = row_v[pl.ds(mbase, 16)] - base_row
            vv = valv[pl.ds(mbase, 16)]
            rv = jnp.where((rv >= 0) & (rv < RPW), rv, RPW)
            r15 = rv[15]
            uniform = (rv[0] == pr) & (r15 == pr) & (pr < RPW)

            @pl.when(uniform)
            def _():
                a0 = acc_v[pr, pl.ds(0, 16)]
                a1 = acc_v[pr, pl.ds(16, 16)]
                a2 = acc_v[pr, pl.ds(32, 16)]
                a3 = acc_v[pr, pl.ds(48, 16)]
                for k in range(16):
                    v = vv[k]
                    a0 = a0 + v * rows_v[rbase + k, pl.ds(0, 16)]
                    a1 = a1 + v * rows_v[rbase + k, pl.ds(16, 16)]
                    a2 = a2 + v * rows_v[rbase + k, pl.ds(32, 16)]
                    a3 = a3 + v * rows_v[rbase + k, pl.ds(48, 16)]
                acc_v[pr, pl.ds(0, 16)] = a0
                acc_v[pr, pl.ds(16, 16)] = a1
                acc_v[pr, pl.ds(32, 16)] = a2
                acc_v[pr, pl.ds(48, 16)] = a3

            @pl.when(jnp.logical_not(uniform))
            def _():
                for k in range(16):
                    r = rv[k]
                    v = vv[k]
                    acc_v[r, pl.ds(0, 16)] += v * rows_v[rbase + k, pl.ds(0, 16)]
                    acc_v[r, pl.ds(16, 16)] += v * rows_v[rbase + k, pl.ds(16, 16)]
                    acc_v[r, pl.ds(32, 16)] += v * rows_v[rbase + k, pl.ds(32, 16)]
                    acc_v[r, pl.ds(48, 16)] += v * rows_v[rbase + k, pl.ds(48, 16)]

            return r15

        return lax.fori_loop(0, H // 16, per_group, pr0)

    @pl.when(nch > 0)
    def _():
        g0 = pl.multiple_of(start, GK)
        pltpu.sync_copy(col_hbm.at[pl.ds(g0, CK)], idx_v.at[pl.ds(0, CK)])
        pltpu.sync_copy(row_hbm.at[pl.ds(g0, CK)], row_v.at[pl.ds(0, CK)])
        pltpu.sync_copy(val_hbm.at[pl.ds(g0, CK)], valv.at[pl.ds(0, CK)])
        issue_wave(0, 0, gsem)

        @pl.when(nch > 1)
        def _():
            issue_meta(1, CK)

        def do_chunk(ci, pr):
            slot = lax.rem(ci, 2)
            so = pl.multiple_of(slot * CK, CK)
            nso = pl.multiple_of(CK - slot * CK, CK)
            issue_wave(so, 1, gsem2)
            wait_wave(gsem)
            pr = compute_half(so, 0, pr)

            @pl.when(ci + 1 < nch)
            def _():
                wait_meta()
                issue_wave(nso, 0, gsem)

            wait_wave(gsem2)
            pr = compute_half(so, H, pr)

            @pl.when(ci + 2 < nch)
            def _():
                issue_meta(ci + 2, so)

            return pr

        lax.fori_loop(0, nch, do_chunk, jnp.int32(RPW))

    def finish_row(r, carry):
        for j in range(OUTPUT_DIM // 16):
            x = acc_v[r, pl.ds(16 * j, 16)] + bias_v[pl.ds(16 * j, 16)]
            acc_v[r, pl.ds(16 * j, 16)] = jnp.maximum(x, 0.0)
        return carry

    lax.fori_loop(0, RPW, finish_row, 0)
    pltpu.sync_copy(acc_v.at[pl.ds(0, RPW)], out_hbm.at[pl.ds(base_row, RPW)])


def kernel(values, row_idx, col_idx, weights, bias):
    edges = jnp.arange(RPW, BATCH, RPW, dtype=jnp.int32)
    inner = jnp.sum(row_idx[None, :] < edges[:, None], axis=1, dtype=jnp.int32)
    bounds = jnp.concatenate(
        [jnp.zeros((1,), jnp.int32), inner, jnp.full((1,), NNZ, jnp.int32)]
    )
    cs = (bounds[:-1] // GK) * GK
    ce = ((bounds[1:] + (GK - 1)) // GK) * GK
    nch = (ce - cs + (CK - 1)) // CK
    cs = jnp.pad(cs, (0, 16))
    nch = jnp.pad(nch, (0, 16))
    values = jnp.concatenate([values, jnp.zeros((CK,), values.dtype)])
    row_idx = jnp.concatenate([row_idx, jnp.full((CK,), BATCH - 1, row_idx.dtype)])
    col_idx = jnp.concatenate([col_idx, jnp.zeros((CK,), col_idx.dtype)])
    return _embedding_bag(values, row_idx, col_idx, weights, bias, cs, nch)


# speedup vs baseline: 2.0114x; 1.1903x over previous
import functools

import numpy as np

import jax
import jax.numpy as jnp
from jax import lax
from jax.experimental import pallas as pl
from jax.experimental.pallas import tpu as pltpu
from jax.experimental.pallas import tpu_sc as plsc

BATCH = 4096
INPUT_DIM = 100000
OUTPUT_DIM = 64
NNZ = 204800

NC, NS = 2, 16
NW = NC * NS
RPW = BATCH // NW
GK = 128
NG = 8
CK = GK * NG

_mesh = plsc.VectorSubcoreMesh(
    core_axis_name="c", subcore_axis_name="s", num_cores=NC, num_subcores=NS
)


@functools.partial(
    pl.kernel,
    out_type=jax.ShapeDtypeStruct((BATCH, OUTPUT_DIM), jnp.float32),
    mesh=_mesh,
    scratch_types=[
        pltpu.VMEM((NW + 16,), jnp.int32),
        pltpu.VMEM((NW + 16,), jnp.int32),
        pltpu.VMEM((OUTPUT_DIM,), jnp.float32),
        pltpu.VMEM((2 * CK,), jnp.int32),
        pltpu.VMEM((2 * CK,), jnp.int32),
        pltpu.VMEM((2 * CK,), jnp.float32),
        pltpu.VMEM((CK, OUTPUT_DIM), jnp.float32),
        pltpu.VMEM((RPW + 1, OUTPUT_DIM), jnp.float32),
        pltpu.SemaphoreType.DMA,
        pltpu.SemaphoreType.DMA,
        pltpu.SemaphoreType.DMA,
    ],
    compiler_params=pltpu.CompilerParams(
        use_tc_tiling_on_sc=False, needs_layout_passes=False
    ),
)
def _embedding_bag(val_hbm, row_hbm, col_hbm, w_hbm, bias_hbm, cs_hbm, nch_hbm,
                   out_hbm, cs_v, nch_v, bias_v, idx_v, row_v, valv, rows_v,
                   acc_v, gsem, gsem2, msem):
    wid = lax.axis_index("s") * NC + lax.axis_index("c")
    base_row = wid * RPW

    pltpu.sync_copy(cs_hbm, cs_v)
    pltpu.sync_copy(nch_hbm, nch_v)
    pltpu.sync_copy(bias_hbm, bias_v)
    start = cs_v[pl.ds(wid, 16)][0]
    nch = nch_v[pl.ds(wid, 16)][0]

    zeros = jnp.zeros((16,), jnp.float32)

    def zero_row(r, carry):
        for j in range(OUTPUT_DIM // 16):
            acc_v[r, pl.ds(16 * j, 16)] = zeros
        return carry

    lax.fori_loop(0, RPW + 1, zero_row, 0)

    H = CK // 2
    NGH = NG // 2

    def issue_meta(ci, slot_off):
        g = pl.multiple_of(start + ci * CK, GK)
        so = pl.multiple_of(slot_off, CK)
        pltpu.async_copy(col_hbm.at[pl.ds(g, CK)], idx_v.at[pl.ds(so, CK)], msem)
        pltpu.async_copy(row_hbm.at[pl.ds(g, CK)], row_v.at[pl.ds(so, CK)], msem)
        pltpu.async_copy(val_hbm.at[pl.ds(g, CK)], valv.at[pl.ds(so, CK)], msem)

    def wait_meta():
        pltpu.make_async_copy(col_hbm.at[pl.ds(0, CK)], idx_v.at[pl.ds(0, CK)], msem).wait()
        pltpu.make_async_copy(row_hbm.at[pl.ds(0, CK)], row_v.at[pl.ds(0, CK)], msem).wait()
        pltpu.make_async_copy(val_hbm.at[pl.ds(0, CK)], valv.at[pl.ds(0, CK)], msem).wait()

    def issue_wave(slot_off, wave, sem):
        for j in range(NGH):
            off = wave * H + j * GK
            pltpu.async_copy(
                w_hbm.at[idx_v.at[pl.ds(pl.multiple_of(slot_off + off, GK), GK)]],
                rows_v.at[pl.ds(off, GK)],
                sem,
            )

    def wait_wave(sem):
        pltpu.make_async_copy(
            w_hbm.at[pl.ds(0, H)], rows_v.at[pl.ds(0, H)], sem
        ).wait()

    iota16 = lax.iota(jnp.int32, 16)
    zeros_i = jnp.zeros((16,), jnp.int32)

    def compute_half(slot_off, rows_off, st0):
        def per_group(i16, st):
            rbase = rows_off + i16 * 16
            mbase = slot_off + rbase
            rv = row_v[pl.ds(mbase, 16)] - base_row
            vv = valv[pl.ds(mbase, 16)]
            rvc = jnp.where((rv >= 0) & (rv < RPW), rv, RPW)
            pr0 = st[0]
            uniform = (rvc[0] == pr0) & (rvc[15] == pr0) & (pr0 < RPW)

            def fast(st):
                pr, a0, a1, a2, a3 = st
                for k in range(16):
                    v = vv[k]
                    a0 = a0 + v * rows_v[rbase + k, pl.ds(0, 16)]
                    a1 = a1 + v * rows_v[rbase + k, pl.ds(16, 16)]
                    a2 = a2 + v * rows_v[rbase + k, pl.ds(32, 16)]
                    a3 = a3 + v * rows_v[rbase + k, pl.ds(48, 16)]
                return pr, a0, a1, a2, a3

            def slow(st):
                pr, a0, a1, a2, a3 = st
                for k in range(16):
                    r = rvc[k]
                    v = vv[k]
                    new = r != pr
                    newi = jnp.where(new, 1, 0)
                    m = (zeros_i + newi) > 0
                    prv = zeros_i + pr
                    for j, aj in enumerate((a0, a1, a2, a3)):
                        plsc.addupdate_scatter(
                            acc_v, [prv, iota16 + 16 * j], aj, mask=m
                        )
                    keep = jnp.where(new, 0.0, 1.0)
                    a0 = a0 * keep + v * rows_v[rbase + k, pl.ds(0, 16)]
                    a1 = a1 * keep + v * rows_v[rbase + k, pl.ds(16, 16)]
                    a2 = a2 * keep + v * rows_v[rbase + k, pl.ds(32, 16)]
                    a3 = a3 * keep + v * rows_v[rbase + k, pl.ds(48, 16)]
                    pr = r
                return pr, a0, a1, a2, a3

            return lax.cond(uniform, fast, slow, st)

        return lax.fori_loop(0, H // 16, per_group, st0)

    @pl.when(nch > 0)
    def _():
        g0 = pl.multiple_of(start, GK)
        pltpu.sync_copy(col_hbm.at[pl.ds(g0, CK)], idx_v.at[pl.ds(0, CK)])
        pltpu.sync_copy(row_hbm.at[pl.ds(g0, CK)], row_v.at[pl.ds(0, CK)])
        pltpu.sync_copy(val_hbm.at[pl.ds(g0, CK)], valv.at[pl.ds(0, CK)])
        issue_wave(0, 0, gsem)

        @pl.when(nch > 1)
        def _():
            issue_meta(1, CK)

        def do_chunk(ci, carry):
            slot = lax.rem(ci, 2)
            so = pl.multiple_of(slot * CK, CK)
            nso = pl.multiple_of(CK - slot * CK, CK)
            issue_wave(so, 1, gsem2)
            wait_wave(gsem)
            st = compute_half(so, 0, (jnp.int32(RPW), zeros, zeros, zeros, zeros))

            @pl.when(ci + 1 < nch)
            def _():
                wait_meta()
                issue_wave(nso, 0, gsem)

            wait_wave(gsem2)
            st = compute_half(so, H, st)

            @pl.when(ci + 2 < nch)
            def _():
                issue_meta(ci + 2, so)

            pr, *regs = st
            for j in range(OUTPUT_DIM // 16):
                plsc.addupdate(acc_v.at[pr, pl.ds(16 * j, 16)], regs[j])
            return carry

        lax.fori_loop(0, nch, do_chunk, 0)

    def finish_row(r, carry):
        for j in range(OUTPUT_DIM // 16):
            x = acc_v[r, pl.ds(16 * j, 16)] + bias_v[pl.ds(16 * j, 16)]
            acc_v[r, pl.ds(16 * j, 16)] = jnp.maximum(x, 0.0)
        return carry

    lax.fori_loop(0, RPW, finish_row, 0)
    pltpu.sync_copy(acc_v.at[pl.ds(0, RPW)], out_hbm.at[pl.ds(base_row, RPW)])


def kernel(values, row_idx, col_idx, weights, bias):
    edges = jnp.arange(RPW, BATCH, RPW, dtype=jnp.int32)
    inner = jnp.sum(row_idx[None, :] < edges[:, None], axis=1, dtype=jnp.int32)
    bounds = jnp.concatenate(
        [jnp.zeros((1,), jnp.int32), inner, jnp.full((1,), NNZ, jnp.int32)]
    )
    cs = (bounds[:-1] // GK) * GK
    ce = ((bounds[1:] + (GK - 1)) // GK) * GK
    nch = (ce - cs + (CK - 1)) // CK
    cs = jnp.pad(cs, (0, 16))
    nch = jnp.pad(nch, (0, 16))
    values = jnp.concatenate([values, jnp.zeros((CK,), values.dtype)])
    row_idx = jnp.concatenate([row_idx, jnp.full((CK,), BATCH - 1, row_idx.dtype)])
    col_idx = jnp.concatenate([col_idx, jnp.zeros((CK,), col_idx.dtype)])
    return _embedding_bag(values, row_idx, col_idx, weights, bias, cs, nch)


# speedup vs baseline: 2.2071x; 1.0973x over previous
import functools

import numpy as np

import jax
import jax.numpy as jnp
from jax import lax
from jax.experimental import pallas as pl
from jax.experimental.pallas import tpu as pltpu
from jax.experimental.pallas import tpu_sc as plsc

BATCH = 4096
INPUT_DIM = 100000
OUTPUT_DIM = 64
NNZ = 204800

NC, NS = 2, 16
NW = NC * NS
RPW = BATCH // NW
GK = 128
NG = 4
CK = GK * NG

_mesh = plsc.VectorSubcoreMesh(
    core_axis_name="c", subcore_axis_name="s", num_cores=NC, num_subcores=NS
)


@functools.partial(
    pl.kernel,
    out_type=jax.ShapeDtypeStruct((BATCH, OUTPUT_DIM), jnp.float32),
    mesh=_mesh,
    scratch_types=[
        pltpu.VMEM((NW + 16,), jnp.int32),
        pltpu.VMEM((NW + 16,), jnp.int32),
        pltpu.VMEM((OUTPUT_DIM,), jnp.float32),
        pltpu.VMEM((2 * CK,), jnp.int32),
        pltpu.VMEM((2 * CK,), jnp.int32),
        pltpu.VMEM((2 * CK,), jnp.float32),
        pltpu.VMEM((CK, OUTPUT_DIM), jnp.float32),
        pltpu.VMEM((RPW + 1, OUTPUT_DIM), jnp.float32),
        pltpu.SemaphoreType.DMA,
        pltpu.SemaphoreType.DMA,
        pltpu.SemaphoreType.DMA,
    ],
    compiler_params=pltpu.CompilerParams(
        use_tc_tiling_on_sc=False, needs_layout_passes=False
    ),
)
def _embedding_bag(val_hbm, row_hbm, col_hbm, w_hbm, bias_hbm, cs_hbm, nch_hbm,
                   out_hbm, cs_v, nch_v, bias_v, idx_v, row_v, valv, rows_v,
                   acc_v, gsem, gsem2, msem):
    wid = lax.axis_index("s") * NC + lax.axis_index("c")
    base_row = wid * RPW

    pltpu.sync_copy(cs_hbm, cs_v)
    pltpu.sync_copy(nch_hbm, nch_v)
    pltpu.sync_copy(bias_hbm, bias_v)
    start = cs_v[pl.ds(wid, 16)][0]
    nch = nch_v[pl.ds(wid, 16)][0]

    zeros = jnp.zeros((16,), jnp.float32)

    def zero_row(r, carry):
        for j in range(OUTPUT_DIM // 16):
            acc_v[r, pl.ds(16 * j, 16)] = zeros
        return carry

    lax.fori_loop(0, RPW + 1, zero_row, 0)

    H = CK // 2
    NGH = NG // 2

    def issue_meta(ci, slot_off):
        g = pl.multiple_of(start + ci * CK, GK)
        so = pl.multiple_of(slot_off, CK)
        pltpu.async_copy(col_hbm.at[pl.ds(g, CK)], idx_v.at[pl.ds(so, CK)], msem)
        pltpu.async_copy(row_hbm.at[pl.ds(g, CK)], row_v.at[pl.ds(so, CK)], msem)
        pltpu.async_copy(val_hbm.at[pl.ds(g, CK)], valv.at[pl.ds(so, CK)], msem)

    def wait_meta():
        pltpu.make_async_copy(col_hbm.at[pl.ds(0, CK)], idx_v.at[pl.ds(0, CK)], msem).wait()
        pltpu.make_async_copy(row_hbm.at[pl.ds(0, CK)], row_v.at[pl.ds(0, CK)], msem).wait()
        pltpu.make_async_copy(val_hbm.at[pl.ds(0, CK)], valv.at[pl.ds(0, CK)], msem).wait()

    def issue_wave(slot_off, wave, sem):
        for j in range(NGH):
            off = wave * H + j * GK
            pltpu.async_copy(
                w_hbm.at[idx_v.at[pl.ds(pl.multiple_of(slot_off + off, GK), GK)]],
                rows_v.at[pl.ds(off, GK)],
                sem,
            )

    def wait_wave(sem):
        pltpu.make_async_copy(
            w_hbm.at[pl.ds(0, H)], rows_v.at[pl.ds(0, H)], sem
        ).wait()

    iota16 = lax.iota(jnp.int32, 16)
    zeros_i = jnp.zeros((16,), jnp.int32)

    def compute_half(slot_off, rows_off, st0):
        def per_group(i16, st):
            rbase = rows_off + i16 * 16
            mbase = slot_off + rbase
            rv = row_v[pl.ds(mbase, 16)] - base_row
            vv = valv[pl.ds(mbase, 16)]
            rvc = jnp.where((rv >= 0) & (rv < RPW), rv, RPW)
            pr0 = st[0]
            uniform = (rvc[0] == pr0) & (rvc[15] == pr0) & (pr0 < RPW)

            def fast(st):
                pr, a0, a1, a2, a3 = st
                for k in range(16):
                    v = vv[k]
                    a0 = a0 + v * rows_v[rbase + k, pl.ds(0, 16)]
                    a1 = a1 + v * rows_v[rbase + k, pl.ds(16, 16)]
                    a2 = a2 + v * rows_v[rbase + k, pl.ds(32, 16)]
                    a3 = a3 + v * rows_v[rbase + k, pl.ds(48, 16)]
                return pr, a0, a1, a2, a3

            def slow(st):
                pr, a0, a1, a2, a3 = st
                for k in range(16):
                    r = rvc[k]
                    v = vv[k]
                    new = r != pr
                    newi = jnp.where(new, 1, 0)
                    m = (zeros_i + newi) > 0
                    prv = zeros_i + pr
                    for j, aj in enumerate((a0, a1, a2, a3)):
                        plsc.addupdate_scatter(
                            acc_v, [prv, iota16 + 16 * j], aj, mask=m
                        )
                    keep = jnp.where(new, 0.0, 1.0)
                    a0 = a0 * keep + v * rows_v[rbase + k, pl.ds(0, 16)]
                    a1 = a1 * keep + v * rows_v[rbase + k, pl.ds(16, 16)]
                    a2 = a2 * keep + v * rows_v[rbase + k, pl.ds(32, 16)]
                    a3 = a3 * keep + v * rows_v[rbase + k, pl.ds(48, 16)]
                    pr = r
                return pr, a0, a1, a2, a3

            return lax.cond(uniform, fast, slow, st)

        return lax.fori_loop(0, H // 16, per_group, st0)

    @pl.when(nch > 0)
    def _():
        g0 = pl.multiple_of(start, GK)
        pltpu.sync_copy(col_hbm.at[pl.ds(g0, CK)], idx_v.at[pl.ds(0, CK)])
        pltpu.sync_copy(row_hbm.at[pl.ds(g0, CK)], row_v.at[pl.ds(0, CK)])
        pltpu.sync_copy(val_hbm.at[pl.ds(g0, CK)], valv.at[pl.ds(0, CK)])
        issue_wave(0, 0, gsem)

        @pl.when(nch > 1)
        def _():
            issue_meta(1, CK)

        def do_chunk(ci, carry):
            slot = lax.rem(ci, 2)
            so = pl.multiple_of(slot * CK, CK)
            nso = pl.multiple_of(CK - slot * CK, CK)
            issue_wave(so, 1, gsem2)
            wait_wave(gsem)
            st = compute_half(so, 0, (jnp.int32(RPW), zeros, zeros, zeros, zeros))

            @pl.when(ci + 1 < nch)
            def _():
                wait_meta()
                issue_wave(nso, 0, gsem)

            wait_wave(gsem2)
            st = compute_half(so, H, st)

            @pl.when(ci + 2 < nch)
            def _():
                issue_meta(ci + 2, so)

            pr, *regs = st
            for j in range(OUTPUT_DIM // 16):
                plsc.addupdate(acc_v.at[pr, pl.ds(16 * j, 16)], regs[j])
            return carry

        lax.fori_loop(0, nch, do_chunk, 0)

    def finish_row(r, carry):
        for j in range(OUTPUT_DIM // 16):
            x = acc_v[r, pl.ds(16 * j, 16)] + bias_v[pl.ds(16 * j, 16)]
            acc_v[r, pl.ds(16 * j, 16)] = jnp.maximum(x, 0.0)
        return carry

    lax.fori_loop(0, RPW, finish_row, 0)
    pltpu.sync_copy(acc_v.at[pl.ds(0, RPW)], out_hbm.at[pl.ds(base_row, RPW)])


def kernel(values, row_idx, col_idx, weights, bias):
    edges = jnp.arange(RPW, BATCH, RPW, dtype=jnp.int32)
    inner = jnp.sum(row_idx[None, :] < edges[:, None], axis=1, dtype=jnp.int32)
    bounds = jnp.concatenate(
        [jnp.zeros((1,), jnp.int32), inner, jnp.full((1,), NNZ, jnp.int32)]
    )
    cs = (bounds[:-1] // GK) * GK
    ce = ((bounds[1:] + (GK - 1)) // GK) * GK
    nch = (ce - cs + (CK - 1)) // CK
    cs = jnp.pad(cs, (0, 16))
    nch = jnp.pad(nch, (0, 16))
    values = jnp.concatenate([values, jnp.zeros((CK,), values.dtype)])
    row_idx = jnp.concatenate([row_idx, jnp.full((CK,), BATCH - 1, row_idx.dtype)])
    col_idx = jnp.concatenate([col_idx, jnp.zeros((CK,), col_idx.dtype)])
    return _embedding_bag(values, row_idx, col_idx, weights, bias, cs, nch)
